# Initial kernel scaffold; baseline (speedup 1.0000x reference)
import jax
import jax.numpy as jnp
from jax.experimental import pallas as pl


def kernel(indices):
    raise NotImplementedError("write your pallas kernel here")


# speedup vs baseline: 1.0772x; 1.0772x over previous
import functools

import jax
import jax.numpy as jnp
from jax import lax
from jax.experimental import pallas as pl
from jax.experimental.pallas import tpu as pltpu
from jax.experimental.pallas import tpu_sc as plsc

NUM_ROWS = 16384
NUM_COLS = 1000

NC = 2
NS = 16
L = 16
NW = NC * NS

ROWS_PER_W = NUM_ROWS // NW
CHUNK_ROWS = 128
NCHUNK = ROWS_PER_W // CHUNK_ROWS
CHUNK_ELEMS = CHUNK_ROWS * NUM_COLS


def _body(idx_hbm, out_hbm, idx_v, buf):
    wid = lax.axis_index("s") * NC + lax.axis_index("c")
    base = wid * ROWS_PER_W

    pltpu.sync_copy(idx_hbm.at[pl.ds(base * 1, ROWS_PER_W)], idx_v)

    zeros16 = jnp.zeros((L,), jnp.float32)

    def _zero(i, carry):
        for u in range(16):
            buf[pl.ds(i * 256 + u * L, L)] = zeros16
        return carry

    lax.fori_loop(0, CHUNK_ELEMS // 256, _zero, 0)

    lanes = lax.iota(jnp.int32, L)
    ones16 = jnp.ones((L,), jnp.float32)

    for c in range(NCHUNK):
        for g in range(CHUNK_ROWS // L):
            cols = idx_v[pl.ds(c * CHUNK_ROWS + g * L, L)]
            flat = (lanes + g * L) * NUM_COLS + cols
            plsc.store_scatter(buf, [flat], ones16)
        pltpu.sync_copy(
            buf, out_hbm.at[pl.ds((base + c * CHUNK_ROWS) * NUM_COLS, CHUNK_ELEMS)]
        )
        for g in range(CHUNK_ROWS // L):
            cols = idx_v[pl.ds(c * CHUNK_ROWS + g * L, L)]
            flat = (lanes + g * L) * NUM_COLS + cols
            plsc.store_scatter(buf, [flat], zeros16)


@jax.jit
def kernel(indices):
    flat_idx = indices.reshape(NUM_ROWS).astype(jnp.int32)
    mesh = plsc.VectorSubcoreMesh(core_axis_name="c", subcore_axis_name="s")
    out = pl.kernel(
        _body,
        mesh=mesh,
        compiler_params=pltpu.CompilerParams(needs_layout_passes=False),
        out_type=jax.ShapeDtypeStruct((NUM_ROWS * NUM_COLS,), jnp.float32),
        scratch_types=[
            pltpu.VMEM((ROWS_PER_W,), jnp.int32),
            pltpu.VMEM((CHUNK_ELEMS,), jnp.float32),
        ],
    )(flat_idx)
    return out.reshape(NUM_ROWS, NUM_COLS)


# speedup vs baseline: 1.7483x; 1.6229x over previous
import functools

import jax
import jax.numpy as jnp
from jax import lax
from jax.experimental import pallas as pl
from jax.experimental.pallas import tpu as pltpu
from jax.experimental.pallas import tpu_sc as plsc

NUM_ROWS = 16384
NUM_COLS = 1000

NC = 2
NS = 16
L = 16
NW = NC * NS

ROWS_PER_W = NUM_ROWS // NW
CHUNK_ROWS = 64
NCHUNK = ROWS_PER_W // CHUNK_ROWS


def _body(idx_hbm, out_hbm, idx_v, buf):
    wid = lax.axis_index("s") * NC + lax.axis_index("c")
    base = wid * ROWS_PER_W

    pltpu.sync_copy(idx_hbm.at[pl.ds(base * 1, ROWS_PER_W)], idx_v)

    zeros16 = jnp.zeros((L,), jnp.float32)

    def _zero(i, carry):
        for u in range(NUM_COLS // L):
            buf[i, pl.ds(u * L, L)] = zeros16
        buf[i, pl.ds(NUM_COLS - L, L)] = zeros16
        return carry

    lax.fori_loop(0, CHUNK_ROWS, _zero, 0)

    lanes = lax.iota(jnp.int32, L)
    ones16 = jnp.ones((L,), jnp.float32)

    for c in range(NCHUNK):
        for g in range(CHUNK_ROWS // L):
            cols = idx_v[pl.ds(c * CHUNK_ROWS + g * L, L)]
            rows = lanes + g * L
            plsc.store_scatter(buf, [rows, cols], ones16)
        pltpu.sync_copy(buf, out_hbm.at[pl.ds(base + c * CHUNK_ROWS, CHUNK_ROWS)])
        for g in range(CHUNK_ROWS // L):
            cols = idx_v[pl.ds(c * CHUNK_ROWS + g * L, L)]
            rows = lanes + g * L
            plsc.store_scatter(buf, [rows, cols], zeros16)


@jax.jit
def kernel(indices):
    flat_idx = indices.reshape(NUM_ROWS).astype(jnp.int32)
    mesh = plsc.VectorSubcoreMesh(core_axis_name="c", subcore_axis_name="s")
    return pl.kernel(
        _body,
        mesh=mesh,
        compiler_params=pltpu.CompilerParams(needs_layout_passes=False),
        out_type=jax.ShapeDtypeStruct((NUM_ROWS, NUM_COLS), jnp.float32),
        scratch_types=[
            pltpu.VMEM((ROWS_PER_W,), jnp.int32),
            pltpu.VMEM((CHUNK_ROWS, NUM_COLS), jnp.float32),
        ],
    )(flat_idx)


# speedup vs baseline: 3.9280x; 2.2468x over previous
import functools

import jax
import jax.numpy as jnp
from jax import lax
from jax.experimental import pallas as pl
from jax.experimental.pallas import tpu as pltpu
from jax.experimental.pallas import tpu_sc as plsc

NUM_ROWS = 16384
NUM_COLS = 1000

NC = 2
NS = 16
L = 16
NW = NC * NS

COLS_PER_W = NUM_ROWS // NW
CHUNK_COLS = 128
NCHUNK = COLS_PER_W // CHUNK_COLS


def _body(idx_hbm, out_hbm, idx_v, buf):
    wid = lax.axis_index("s") * NC + lax.axis_index("c")
    base = wid * COLS_PER_W

    pltpu.sync_copy(idx_hbm.at[pl.ds(base * 1, COLS_PER_W)], idx_v)

    zeros16 = jnp.zeros((L,), jnp.float32)

    def _zero(i, carry):
        for u in range(CHUNK_COLS // L):
            buf[i, pl.ds(u * L, L)] = zeros16
        return carry

    lax.fori_loop(0, NUM_COLS, _zero, 0)

    lanes = lax.iota(jnp.int32, L)
    ones16 = jnp.ones((L,), jnp.float32)

    for c in range(NCHUNK):
        for g in range(CHUNK_COLS // L):
            rows = idx_v[pl.ds(c * CHUNK_COLS + g * L, L)]
            cols = lanes + g * L
            plsc.store_scatter(buf, [rows, cols], ones16)
        pltpu.sync_copy(
            buf, out_hbm.at[:, pl.ds(base + c * CHUNK_COLS, CHUNK_COLS)]
        )
        for g in range(CHUNK_COLS // L):
            rows = idx_v[pl.ds(c * CHUNK_COLS + g * L, L)]
            cols = lanes + g * L
            plsc.store_scatter(buf, [rows, cols], zeros16)


@jax.jit
def kernel(indices):
    flat_idx = indices.reshape(NUM_ROWS).astype(jnp.int32)
    mesh = plsc.VectorSubcoreMesh(core_axis_name="c", subcore_axis_name="s")
    out_t = pl.kernel(
        _body,
        mesh=mesh,
        compiler_params=pltpu.CompilerParams(needs_layout_passes=False),
        out_type=jax.ShapeDtypeStruct((NUM_COLS, NUM_ROWS), jnp.float32),
        scratch_types=[
            pltpu.VMEM((COLS_PER_W,), jnp.int32),
            pltpu.VMEM((NUM_COLS, CHUNK_COLS), jnp.float32),
        ],
    )(flat_idx)
    return out_t.T


# speedup vs baseline: 4.0587x; 1.0333x over previous
import functools

import jax
import jax.numpy as jnp
from jax import lax
from jax.experimental import pallas as pl
from jax.experimental.pallas import tpu as pltpu
from jax.experimental.pallas import tpu_sc as plsc

NUM_ROWS = 16384
NUM_COLS = 1000

NC = 2
NS = 16
L = 16
NW = NC * NS

COLS_PER_W = NUM_ROWS // NW
CHUNK_COLS = 128
NCHUNK = COLS_PER_W // CHUNK_COLS
SPLIT = 512


def _body(idx_hbm, out_hbm, idx_v, buf_a, buf_b, sem_a, sem_b):
    wid = lax.axis_index("s") * NC + lax.axis_index("c")
    base = wid * COLS_PER_W

    pltpu.sync_copy(idx_hbm.at[pl.ds(base * 1, COLS_PER_W)], idx_v)

    zeros16 = jnp.zeros((L,), jnp.float32)
    ones16 = jnp.ones((L,), jnp.float32)
    lanes = lax.iota(jnp.int32, L)

    def _zero(buf, nrows):
        def body(i, carry):
            for u in range(CHUNK_COLS // L):
                buf[i, pl.ds(u * L, L)] = zeros16
            return carry

        lax.fori_loop(0, nrows, body, 0)

    def _scatter(buf, lo, c, val):
        for g in range(CHUNK_COLS // L):
            rows = idx_v[pl.ds(c * CHUNK_COLS + g * L, L)]
            cols = lanes + g * L
            if lo == 0:
                mask = rows < SPLIT
                plsc.store_scatter(buf, [rows, cols], val, mask=mask)
            else:
                mask = rows >= SPLIT
                plsc.store_scatter(buf, [rows - SPLIT, cols], val, mask=mask)

    def _dst(lo, nrows, c):
        return out_hbm.at[
            pl.ds(lo, nrows), pl.ds(base + c * CHUNK_COLS, CHUNK_COLS)
        ]

    halves = (
        (buf_a, sem_a, 0, SPLIT),
        (buf_b, sem_b, SPLIT, NUM_COLS - SPLIT),
    )

    for buf, sem, lo, nrows in halves:
        _zero(buf, nrows)
        _scatter(buf, lo, 0, ones16)
        pltpu.async_copy(buf, _dst(lo, nrows, 0), sem)
    for c in range(1, NCHUNK):
        for buf, sem, lo, nrows in halves:
            pltpu.make_async_copy(buf, _dst(lo, nrows, c - 1), sem).wait()
            _scatter(buf, lo, c - 1, zeros16)
            _scatter(buf, lo, c, ones16)
            pltpu.async_copy(buf, _dst(lo, nrows, c), sem)
    for buf, sem, lo, nrows in halves:
        pltpu.make_async_copy(buf, _dst(lo, nrows, NCHUNK - 1), sem).wait()


@jax.jit
def kernel(indices):
    flat_idx = indices.reshape(NUM_ROWS).astype(jnp.int32)
    mesh = plsc.VectorSubcoreMesh(core_axis_name="c", subcore_axis_name="s")
    out_t = pl.kernel(
        _body,
        mesh=mesh,
        compiler_params=pltpu.CompilerParams(needs_layout_passes=False),
        out_type=jax.ShapeDtypeStruct((NUM_COLS, NUM_ROWS), jnp.float32),
        scratch_types=[
            pltpu.VMEM((COLS_PER_W,), jnp.int32),
            pltpu.VMEM((SPLIT, CHUNK_COLS), jnp.float32),
            pltpu.VMEM((NUM_COLS - SPLIT, CHUNK_COLS), jnp.float32),
            pltpu.SemaphoreType.DMA,
            pltpu.SemaphoreType.DMA,
        ],
    )(flat_idx)
    return out_t.T


# speedup vs baseline: 4.1017x; 1.0106x over previous
import functools

import jax
import jax.numpy as jnp
from jax import lax
from jax.experimental import pallas as pl
from jax.experimental.pallas import tpu as pltpu
from jax.experimental.pallas import tpu_sc as plsc

NUM_ROWS = 16384
NUM_COLS = 1000

NC = 2
NS = 16
L = 16
NW = NC * NS

COLS_PER_W = NUM_ROWS // NW
CHUNK_COLS = 128
NCHUNK = COLS_PER_W // CHUNK_COLS
SPLIT = 512


def _body(idx_hbm, out_hbm, idx_v, buf_a, buf_b, sem_a, sem_b):
    wid = lax.axis_index("s") * NC + lax.axis_index("c")
    base = wid * COLS_PER_W

    idx_copy = pltpu.make_async_copy(
        idx_hbm.at[pl.ds(base * 1, COLS_PER_W)], idx_v, sem_a
    )
    idx_copy.start()

    zeros16 = jnp.zeros((L,), jnp.float32)
    ones16 = jnp.ones((L,), jnp.float32)
    lanes = lax.iota(jnp.int32, L)

    def _zero(buf, nrows):
        def body(i, carry):
            for r in range(2):
                for u in range(CHUNK_COLS // L):
                    buf[i * 2 + r, pl.ds(u * L, L)] = zeros16
            return carry

        lax.fori_loop(0, nrows // 2, body, 0)

    def _scatter(buf, lo, c, val):
        for g in range(CHUNK_COLS // L):
            rows = idx_v[pl.ds(c * CHUNK_COLS + g * L, L)]
            cols = lanes + g * L
            if lo == 0:
                mask = rows < SPLIT
                plsc.store_scatter(buf, [rows, cols], val, mask=mask)
            else:
                mask = rows >= SPLIT
                plsc.store_scatter(buf, [rows - SPLIT, cols], val, mask=mask)

    def _dst(lo, nrows, c):
        return out_hbm.at[
            pl.ds(lo, nrows), pl.ds(base + c * CHUNK_COLS, CHUNK_COLS)
        ]

    halves = (
        (buf_a, sem_a, 0, SPLIT),
        (buf_b, sem_b, SPLIT, NUM_COLS - SPLIT),
    )

    first = True
    for buf, sem, lo, nrows in halves:
        _zero(buf, nrows)
        if first:
            idx_copy.wait()
            first = False
        _scatter(buf, lo, 0, ones16)
        pltpu.async_copy(buf, _dst(lo, nrows, 0), sem)
    for c in range(1, NCHUNK):
        for buf, sem, lo, nrows in halves:
            pltpu.make_async_copy(buf, _dst(lo, nrows, c - 1), sem).wait()
            _scatter(buf, lo, c - 1, zeros16)
            _scatter(buf, lo, c, ones16)
            pltpu.async_copy(buf, _dst(lo, nrows, c), sem)
    for buf, sem, lo, nrows in halves:
        pltpu.make_async_copy(buf, _dst(lo, nrows, NCHUNK - 1), sem).wait()


@jax.jit
def kernel(indices):
    flat_idx = indices.reshape(NUM_ROWS).astype(jnp.int32)
    mesh = plsc.VectorSubcoreMesh(core_axis_name="c", subcore_axis_name="s")
    out_t = pl.kernel(
        _body,
        mesh=mesh,
        compiler_params=pltpu.CompilerParams(needs_layout_passes=False),
        out_type=jax.ShapeDtypeStruct((NUM_COLS, NUM_ROWS), jnp.float32),
        scratch_types=[
            pltpu.VMEM((COLS_PER_W,), jnp.int32),
            pltpu.VMEM((SPLIT, CHUNK_COLS), jnp.float32),
            pltpu.VMEM((NUM_COLS - SPLIT, CHUNK_COLS), jnp.float32),
            pltpu.SemaphoreType.DMA,
            pltpu.SemaphoreType.DMA,
        ],
    )(flat_idx)
    return out_t.T


# speedup vs baseline: 4.1904x; 1.0216x over previous
import jax
import jax.numpy as jnp
from jax import lax
from jax.experimental import pallas as pl
from jax.experimental.pallas import tpu as pltpu
from jax.experimental.pallas import tpu_sc as plsc

NUM_ROWS = 16384
NUM_COLS = 1000

NC = 2
NS = 16
L = 16
NW = NC * NS

COLS_PER_W = NUM_ROWS // NW
CHUNK_COLS = 128
NCHUNK = COLS_PER_W // CHUNK_COLS
SPLIT = 512


def _body(idx_hbm, out_hbm, idx_v, buf_a, buf_b, sem_a, sem_b):
    wid = lax.axis_index("s") * NC + lax.axis_index("c")
    base = wid * COLS_PER_W

    idx_copy = pltpu.make_async_copy(
        idx_hbm.at[pl.ds(base, COLS_PER_W)], idx_v, sem_a
    )
    idx_copy.start()

    zeros16 = jnp.zeros((L,), jnp.float32)
    ones16 = jnp.ones((L,), jnp.float32)
    lanes = lax.iota(jnp.int32, L)

    def _zero_rows(buf, r_lo, r_hi):
        def body(i, carry):
            for r in range(2):
                for u in range(CHUNK_COLS // L):
                    buf[r_lo + i * 2 + r, pl.ds(u * L, L)] = zeros16
            return carry

        lax.fori_loop(0, (r_hi - r_lo) // 2, body, 0)

    def _scatter(buf, lo, c, val, slab=None):
        for g in range(CHUNK_COLS // L):
            rows = idx_v[pl.ds(c * CHUNK_COLS + g * L, L)]
            cols = lanes + g * L
            local = rows - lo if lo else rows
            if slab is None:
                s_lo, s_hi = 0, buf.shape[0]
            else:
                s_lo, s_hi = slab
            mask = (local >= s_lo) & (local < s_hi)
            plsc.store_scatter(buf, [local, cols], val, mask=mask)

    def _dst(lo, nrows, c):
        return out_hbm.at[
            pl.ds(lo, nrows), pl.ds(base + c * CHUNK_COLS, CHUNK_COLS)
        ]

    halves = (
        (buf_a, sem_a, 0, SPLIT),
        (buf_b, sem_b, SPLIT, NUM_COLS - SPLIT),
    )

    first = True
    for buf, sem, lo, nrows in halves:
        for s_lo in range(0, nrows, 128):
            s_hi = min(s_lo + 128, nrows)
            _zero_rows(buf, s_lo, s_hi)
            if first:
                idx_copy.wait()
                first = False
            _scatter(buf, lo, 0, ones16, slab=(s_lo, s_hi))
            pltpu.async_copy(
                buf.at[pl.ds(s_lo, s_hi - s_lo)],
                out_hbm.at[
                    pl.ds(lo + s_lo, s_hi - s_lo), pl.ds(base, CHUNK_COLS)
                ],
                sem,
            )
    for c in range(1, NCHUNK):
        for buf, sem, lo, nrows in halves:
            pltpu.make_async_copy(buf, _dst(lo, nrows, c - 1), sem).wait()
            _scatter(buf, lo, c - 1, zeros16)
            _scatter(buf, lo, c, ones16)
            pltpu.async_copy(buf, _dst(lo, nrows, c), sem)
    for buf, sem, lo, nrows in halves:
        pltpu.make_async_copy(buf, _dst(lo, nrows, NCHUNK - 1), sem).wait()


@jax.jit
def kernel(indices):
    flat_idx = indices.reshape(NUM_ROWS).astype(jnp.int32)
    mesh = plsc.VectorSubcoreMesh(core_axis_name="c", subcore_axis_name="s")
    out_t = pl.kernel(
        _body,
        mesh=mesh,
        compiler_params=pltpu.CompilerParams(needs_layout_passes=False),
        out_type=jax.ShapeDtypeStruct((NUM_COLS, NUM_ROWS), jnp.float32),
        scratch_types=[
            pltpu.VMEM((COLS_PER_W,), jnp.int32),
            pltpu.VMEM((SPLIT, CHUNK_COLS), jnp.float32),
            pltpu.VMEM((NUM_COLS - SPLIT, CHUNK_COLS), jnp.float32),
            pltpu.SemaphoreType.DMA,
            pltpu.SemaphoreType.DMA,
        ],
    )(flat_idx)
    return out_t.T
